# baseline (speedup 1.0000x reference)
import functools
import jax
import jax.numpy as jnp
from jax import lax
from jax.experimental import pallas as pl
from jax.experimental.pallas import tpu as pltpu
from jax.experimental.pallas import tpu_sc as plsc


def _make_gather(B, D, NW, G):
    b_per_w = B // NW
    n_g = b_per_w // G

    mesh = plsc.VectorSubcoreMesh(core_axis_name="c", subcore_axis_name="s")

    @functools.partial(
        pl.kernel,
        mesh=mesh,
        compiler_params=pltpu.CompilerParams(use_tc_tiling_on_sc=True),
        out_type=[
            jax.ShapeDtypeStruct((B, D), jnp.float32),
            jax.ShapeDtypeStruct((B, D), jnp.float32),
        ],
        scratch_types=[
            pltpu.VMEM((n_g, G), jnp.int32),
            pltpu.VMEM((2, G, D), jnp.float32),
            pltpu.SemaphoreType.DMA,
            pltpu.SemaphoreType.DMA,
            pltpu.SemaphoreType.DMA,
            pltpu.SemaphoreType.DMA,
        ],
    )
    def k(iw_hbm, cw_hbm, t_hbm, c_hbm, out_i_hbm, out_c_hbm,
          idx_v, rows_v, sem_g0, sem_g1, sem_o0, sem_o1):
        nc = plsc.get_sparse_core_info().num_cores
        wid = lax.axis_index("s") * nc + lax.axis_index("c")
        base = wid * b_per_w
        sem_g = (sem_g0, sem_g1)
        sem_o = (sem_o0, sem_o1)

        for idx_hbm, src_hbm, out_hbm in (
                (iw_hbm, t_hbm, out_i_hbm),
                (cw_hbm, c_hbm, out_c_hbm)):
            for g in range(n_g):
                pltpu.sync_copy(idx_hbm.at[pl.ds(base + g * G, G)],
                                idx_v.at[g])

            def fire(g, p):
                v16 = idx_v[g, pl.ds(0, 16)]
                cps = []
                for j in range(G):
                    r = v16[j] if j < 16 else idx_v[g, pl.ds(16, 16)][j - 16]
                    cps.append(pltpu.async_copy(
                        src_hbm.at[r], rows_v.at[p, j], sem_g[p]))
                return cps

            out_copies = [None, None]
            copies = [None, None]
            copies[0] = fire(0, 0)
            for g in range(n_g):
                p = g & 1
                for cp in copies[p]:
                    cp.wait()
                if g + 1 < n_g:
                    copies[1 - p] = fire(g + 1, 1 - p)
                if out_copies[p] is not None:
                    out_copies[p].wait()
                out_copies[p] = pltpu.async_copy(
                    rows_v.at[p],
                    out_hbm.at[pl.ds(base + g * G, G)],
                    sem_o[p])
            for oc in out_copies:
                if oc is not None:
                    oc.wait()

    return k


def kernel(input_word, context_word, target_table, context_table):
    V, D = target_table.shape
    B = input_word.shape[0]
    gather = _make_gather(B, D, NW=32, G=16)
    out_i, out_c = gather(
        input_word.astype(jnp.int32),
        context_word.astype(jnp.int32),
        target_table,
        context_table,
    )
    return (out_i, out_c)


# speedup vs baseline: 1.6303x; 1.6303x over previous
import functools
import jax
import jax.numpy as jnp
from jax import lax
from jax.experimental import pallas as pl
from jax.experimental.pallas import tpu as pltpu
from jax.experimental.pallas import tpu_sc as plsc

_NBUF = 4


def _make_gather(B, D, NW):
    b_per_w = B // NW
    n_g = b_per_w // 16
    assert n_g >= 4 and n_g % 2 == 0

    mesh = plsc.VectorSubcoreMesh(core_axis_name="c", subcore_axis_name="s")

    @functools.partial(
        pl.kernel,
        mesh=mesh,
        compiler_params=pltpu.CompilerParams(
            use_tc_tiling_on_sc=True, needs_layout_passes=False),
        out_type=[
            jax.ShapeDtypeStruct((B, D), jnp.float32),
            jax.ShapeDtypeStruct((B, D), jnp.float32),
        ],
        scratch_types=[
            pltpu.VMEM((b_per_w,), jnp.int32),
            pltpu.VMEM((b_per_w,), jnp.int32),
            pltpu.VMEM((D, 128), jnp.float32),
            pltpu.VMEM((D, 128), jnp.float32),
            pltpu.VMEM((D, 128), jnp.float32),
            pltpu.VMEM((D, 128), jnp.float32),
            pltpu.VMEM((2, 16, D), jnp.float32),
            pltpu.SemaphoreType.DMA,
            pltpu.SemaphoreType.DMA,
            pltpu.SemaphoreType.DMA,
            pltpu.SemaphoreType.DMA,
            pltpu.SemaphoreType.DMA,
            pltpu.SemaphoreType.DMA,
        ],
    )
    def k(iw_hbm, cw_hbm, tt_hbm, ct_hbm, out_i_hbm, out_c_hbm,
          idx_a, idx_b, tc0, tc1, tc2, tc3, ostage,
          s0, s1, s2, s3, so0, so1):
        nc = plsc.get_sparse_core_info().num_cores
        wid = lax.axis_index("s") * nc + lax.axis_index("c")
        base = wid * b_per_w
        bufs = (tc0, tc1, tc2, tc3)
        sems = (s0, s1, s2, s3)
        osems = (so0, so1)

        pltpu.sync_copy(iw_hbm.at[pl.ds(base, b_per_w)], idx_a)
        pltpu.sync_copy(cw_hbm.at[pl.ds(base, b_per_w)], idx_b)

        ri = lax.iota(jnp.int32, 16)

        for idx_v, src_hbm, out_hbm in ((idx_a, tt_hbm, out_i_hbm),
                                        (idx_b, ct_hbm, out_c_hbm)):
            def fire(i16, j, slot):
                tcoff = lax.shift_right_logical(i16[j], 7) * 128
                tcoff = pl.multiple_of(tcoff, 128)
                pltpu.async_copy(src_hbm.at[:, pl.ds(tcoff, 128)],
                                 bufs[slot], sems[slot])

            def wait_fetch(slot):
                pltpu.make_async_copy(src_hbm.at[:, pl.ds(0, 128)],
                                      bufs[slot], sems[slot]).wait()

            def wait_out(p):
                pltpu.make_async_copy(
                    ostage.at[p], out_hbm.at[pl.ds(base, 16)],
                    osems[p]).wait()

            def group_body(g, i16, nxt16, p, wait_o, last):
                i16m = i16 & 127
                if wait_o:
                    wait_out(p)
                for j in range(16):
                    slot = j % _NBUF
                    wait_fetch(slot)
                    lvec = jnp.take(i16m, jnp.full((16,), j, jnp.int32))
                    for kk in range(D // 16):
                        vals = plsc.load_gather(bufs[slot],
                                                [ri + kk * 16, lvec])
                        ostage[p, j, pl.ds(kk * 16, 16)] = vals
                    if j + _NBUF < 16:
                        fire(i16, j + _NBUF, slot)
                    elif not last:
                        fire(nxt16, (j + _NBUF) % 16, slot)
                pltpu.async_copy(ostage.at[p],
                                 out_hbm.at[pl.ds(base + g * 16, 16)],
                                 osems[p])

            g0 = idx_v[pl.ds(0, 16)]
            for j in range(_NBUF):
                fire(g0, j, j)
            g1 = idx_v[pl.ds(16, 16)]
            group_body(0, g0, g1, 0, wait_o=False, last=False)
            g2 = idx_v[pl.ds(32, 16)]
            group_body(1, g1, g2, 1, wait_o=False, last=False)

            def loop_body(gg, carry):
                g = gg * 2
                i16 = idx_v[pl.ds(g * 16, 16)]
                n16 = idx_v[pl.ds((g + 1) * 16, 16)]
                n216 = idx_v[pl.ds((g + 2) * 16, 16)]
                group_body(g, i16, n16, 0, wait_o=True, last=False)
                group_body(g + 1, n16, n216, 1, wait_o=True, last=False)
                return carry

            lax.fori_loop(1, n_g // 2 - 1, loop_body, None)

            ga = idx_v[pl.ds((n_g - 2) * 16, 16)]
            gb = idx_v[pl.ds((n_g - 1) * 16, 16)]
            group_body(n_g - 2, ga, gb, 0, wait_o=True, last=False)
            group_body(n_g - 1, gb, gb, 1, wait_o=True, last=True)
            wait_out(0)
            wait_out(1)

    return k


def kernel(input_word, context_word, target_table, context_table):
    V, D = target_table.shape
    B = input_word.shape[0]
    gather = _make_gather(B, D, NW=32)
    out_i, out_c = gather(
        input_word.astype(jnp.int32),
        context_word.astype(jnp.int32),
        jnp.swapaxes(target_table, 0, 1),
        jnp.swapaxes(context_table, 0, 1),
    )
    return (out_i, out_c)


# speedup vs baseline: 2.2236x; 1.3639x over previous
import functools
import jax
import jax.numpy as jnp
from jax import lax
from jax.experimental import pallas as pl
from jax.experimental.pallas import tpu as pltpu
from jax.experimental.pallas import tpu_sc as plsc

_NW = 32
_TPW = 244
_LW = _TPW * 128
_COVER = _NW * _LW
_CAP = 704
_LWH = _LW // 2
_IBLK = 4096


def _make_gather(B, D):
    n_iblk = B // _IBLK

    mesh = plsc.VectorSubcoreMesh(core_axis_name="c", subcore_axis_name="s")

    @functools.partial(
        pl.kernel,
        mesh=mesh,
        compiler_params=pltpu.CompilerParams(
            use_tc_tiling_on_sc=True, needs_layout_passes=False),
        out_type=[
            jax.ShapeDtypeStruct((B, D), jnp.float32),
            jax.ShapeDtypeStruct((B, D), jnp.float32),
        ],
        scratch_types=[
            pltpu.VMEM((_IBLK,), jnp.int32),
            pltpu.VMEM((_CAP,), jnp.int32),
            pltpu.VMEM((_CAP,), jnp.int32),
            pltpu.VMEM((_LWH,), jnp.float32),
            pltpu.VMEM((_LWH,), jnp.float32),
            pltpu.VMEM((_CAP, D), jnp.float32),
            pltpu.SemaphoreType.DMA,
            pltpu.SemaphoreType.DMA,
            pltpu.SemaphoreType.DMA,
            pltpu.SemaphoreType.DMA,
        ],
    )
    def k(iw_hbm, cw_hbm, tt_hbm, ct_hbm, out_i_hbm, out_c_hbm,
          iblk, sel_lane, sel_pos, cb0, cb1, stag,
          sem_i, sem_c0, sem_c1, sem_o):
        nc = plsc.get_sparse_core_info().num_cores
        wid = lax.axis_index("s") * nc + lax.axis_index("c")
        lo = wid * _LW
        lo = pl.multiple_of(lo, 128)
        cbufs = (cb0, cb1)
        csems = (sem_c0, sem_c1)
        ri = lax.iota(jnp.int32, 16)

        for idx_hbm, src_hbm, out_hbm in ((iw_hbm, tt_hbm, out_i_hbm),
                                          (cw_hbm, ct_hbm, out_c_hbm)):
            def scan_pass(wlo):
                def blk(b, carry):
                    cnt, gcnt = carry
                    pltpu.sync_copy(idx_hbm.at[pl.ds(b * _IBLK, _IBLK)],
                                    iblk)

                    def grp(g, c2):
                        cnt2, gcnt2 = c2
                        v16 = iblk[pl.ds(g * 16, 16)]
                        m = (v16 >= lo) & (v16 < lo + _LW)
                        mi = m.astype(jnp.int32)
                        incl = plsc.cumsum(mi)
                        rank = gcnt2 + incl - 1
                        am = m & (rank >= wlo) & (rank < wlo + _CAP)
                        plsc.store_compressed(
                            sel_lane.at[pl.ds(cnt2, 16)], v16 - lo, mask=am)
                        plsc.store_compressed(
                            sel_pos.at[pl.ds(cnt2, 16)],
                            b * _IBLK + g * 16 + ri, mask=am)
                        pc_all = plsc.all_reduce_population_count(m)[0]
                        pc_app = plsc.all_reduce_population_count(am)[0]
                        return (cnt2 + pc_app, gcnt2 + pc_all)

                    return lax.fori_loop(0, _IBLK // 16, grp, (cnt, gcnt))

                cnt = jnp.int32(0)
                gcnt = jnp.int32(0)
                for b in range(n_iblk):
                    cnt, gcnt = blk(b, (cnt, gcnt))
                return cnt, gcnt

            def fire_chunk(c, h, p):
                off = lo + h * _LWH
                off = pl.multiple_of(off, 128)
                pltpu.async_copy(src_hbm.at[c, pl.ds(off, _LWH)],
                                 cbufs[p], csems[p])

            def wait_chunk(p):
                pltpu.make_async_copy(src_hbm.at[0, pl.ds(0, _LWH)],
                                      cbufs[p], csems[p]).wait()

            def gather_features(lcnt):
                ng = lax.shift_right_logical(lcnt + 15, 4)

                def gather_half(c, h, p):
                    cvec = ri * 0 + c

                    def grp(g, carry):
                        sl16 = sel_lane[pl.ds(g * 16, 16)] - h * _LWH
                        msk = ((g * 16 + ri) < lcnt) & (sl16 >= 0) \
                            & (sl16 < _LWH)
                        vals = plsc.load_gather(cbufs[p], [sl16], mask=msk)
                        plsc.store_scatter(stag, [g * 16 + ri, cvec],
                                           vals, mask=msk)
                        return carry

                    lax.fori_loop(0, ng, grp, None)

                fire_chunk(0, 0, 0)
                fire_chunk(0, 1, 1)

                def feat(c, carry):
                    wait_chunk(0)
                    gather_half(c, 0, 0)
                    fire_chunk(c + 1, 0, 0)
                    wait_chunk(1)
                    gather_half(c, 1, 1)
                    fire_chunk(c + 1, 1, 1)
                    return carry

                lax.fori_loop(0, D - 1, feat, None)
                wait_chunk(0)
                gather_half(D - 1, 0, 0)
                wait_chunk(1)
                gather_half(D - 1, 1, 1)

            def write_out(lcnt):
                ng = lax.shift_right_logical(lcnt + 15, 4)

                def grp(g, carry):
                    p16 = sel_pos[pl.ds(g * 16, 16)]
                    for j in range(16):
                        @pl.when(g * 16 + j < lcnt)
                        def _():
                            pltpu.async_copy(stag.at[g * 16 + j],
                                             out_hbm.at[p16[j]], sem_o)
                    return carry

                lax.fori_loop(0, ng, grp, None)

                def drain(g, carry):
                    for j in range(16):
                        @pl.when(g * 16 + j < lcnt)
                        def _():
                            pltpu.make_async_copy(
                                stag.at[0], out_hbm.at[0], sem_o).wait()
                    return carry

                lax.fori_loop(0, ng, drain, None)

            def pass_body(carry):
                wlo, _gt = carry
                lcnt, gtotal = scan_pass(wlo)
                gather_features(lcnt)
                write_out(lcnt)
                return (wlo + _CAP, gtotal)

            def pass_cond(carry):
                wlo, gtotal = carry
                return wlo < gtotal

            lax.while_loop(pass_cond, pass_body,
                           (jnp.int32(0), jnp.int32(1)))

    return k


def kernel(input_word, context_word, target_table, context_table):
    V, D = target_table.shape
    B = input_word.shape[0]
    iw = input_word.astype(jnp.int32)
    cw = context_word.astype(jnp.int32)
    gather = _make_gather(B, D)
    out_i, out_c = gather(
        iw, cw,
        jnp.swapaxes(target_table, 0, 1),
        jnp.swapaxes(context_table, 0, 1),
    )
    tail_t = target_table[_COVER:]
    tail_c = context_table[_COVER:]
    fi = jnp.take(tail_t, jnp.clip(iw - _COVER, 0, V - _COVER - 1), axis=0)
    fc = jnp.take(tail_c, jnp.clip(cw - _COVER, 0, V - _COVER - 1), axis=0)
    out_i = jnp.where((iw >= _COVER)[:, None], fi, out_i)
    out_c = jnp.where((cw >= _COVER)[:, None], fc, out_c)
    return (out_i, out_c)
